# baseline (speedup 1.0000x reference)
import functools

import jax
import jax.numpy as jnp
from jax import lax
from jax.experimental import pallas as pl
from jax.experimental.pallas import tpu as pltpu
from jax.experimental.pallas import tpu_sc as plsc

ALPHA = 0.4
SEQ = 2048
FEAT = 768
BATCH = 4
CHUNKS = 4
NC, NS = 2, 16
NW = NC * NS
ROWS_W = SEQ // NW
HALF = ROWS_W // 2
QTR = 16
NT = FEAT // 16
OUT_ROWS = BATCH * CHUNKS * SEQ


def _build_sc_call():
    mesh = plsc.VectorSubcoreMesh(core_axis_name="c", subcore_axis_name="s")

    @functools.partial(
        pl.kernel,
        mesh=mesh,
        out_type=jax.ShapeDtypeStruct((BATCH * CHUNKS, SEQ, FEAT), jnp.float32),
        scratch_types=[
            pltpu.VMEM((HALF, FEAT), jnp.float32),
            pltpu.VMEM((HALF, FEAT), jnp.float32),
            pltpu.VMEM((4, FEAT), jnp.float32),
            pltpu.VMEM((4, FEAT), jnp.float32),
            pltpu.VMEM((2, QTR, FEAT), jnp.float32),
            pltpu.SemaphoreType.DMA,
            pltpu.SemaphoreType.DMA,
            pltpu.SemaphoreType.DMA,
            pltpu.SemaphoreType.DMA,
        ],
    )
    def pe_kernel(table, out, g0, g1, head, cq, bufs, si, s0, sa, sb):
        w = lax.axis_index("s") * NC + lax.axis_index("c")
        j0 = w * ROWS_W
        cp_head = pltpu.async_copy(table.at[pl.ds(0, 4), :], head, si)
        cp_g0 = pltpu.async_copy(table.at[pl.ds(j0, HALF), :], g0, si)
        cp_g1 = pltpu.async_copy(table.at[pl.ds(j0 + HALF, HALF), :], g1, si)

        a = ALPHA
        s = a / (1.0 - a)
        cp_head.wait()
        for t in range(NT):
            sl = pl.ds(t * 16, 16)
            p0 = head[0, sl]
            pq = head[1, sl]
            cq[1, sl] = s * (pq - a * p0) - a * p0

        gsem = (sa, sb)
        pending = {}
        for ph in range(2 * HALF // QTR):
            seg = g0 if ph * QTR < HALF else g1
            r0 = ph * QTR % HALF
            gen = ph % 2
            if ph == 0:
                cp_g0.wait()
            if ph * QTR == HALF:
                cp_g1.wait()
            if ph >= 2:
                pending[ph - 2].wait()
            buf = bufs.at[gen]

            @plsc.parallel_loop(0, NT)
            def body(t, _seg=seg, _buf=buf, _r0=r0):
                sl = pl.ds(t * 16, 16)
                c1 = cq[1, sl]
                for r in range(QTR):
                    _buf[r, sl] = _seg[_r0 + r, sl] + c1

            pending[ph] = pltpu.async_copy(
                buf, out.at[1, pl.ds(j0 + ph * QTR, QTR), :], gsem[gen]
            )
        pending[2].wait()
        pending[3].wait()

    return pe_kernel


def _tc_body(table_ref, big_ref, out_ref):
    k = pl.program_id(0)
    bidx = jnp.where(k == 0, 0, k + 1)
    q = lax.rem(bidx, CHUNKS)
    t = table_ref[...]
    a = ALPHA
    s = a / (1.0 - a)
    p0 = t[0:1, :]
    pq = (
        jnp.where(q == 0, t[0:1, :], 0.0)
        + jnp.where(q == 1, t[1:2, :], 0.0)
        + jnp.where(q == 2, t[2:3, :], 0.0)
        + jnp.where(q == 3, t[3:4, :], 0.0)
    )
    c = s * (pq - a * p0) - a * p0
    out_ref[0] = t + c


def _build_tc_call():
    nblk = BATCH * CHUNKS
    return pl.pallas_call(
        _tc_body,
        grid=(BATCH * CHUNKS - 1,),
        in_specs=[
            pl.BlockSpec((SEQ, FEAT), lambda k: (0, 0)),
            pl.BlockSpec(memory_space=pl.ANY),
        ],
        out_specs=pl.BlockSpec(
            (1, SEQ, FEAT), lambda k: (jnp.where(k == 0, 0, k + 1), 0, 0)
        ),
        out_shape=jax.ShapeDtypeStruct((nblk, SEQ, FEAT), jnp.float32),
        input_output_aliases={1: 0},
    )


_sc_call = _build_sc_call()
_tc_call = _build_tc_call()


@jax.jit
def _pe_full(position_embeddings):
    sc_out = _sc_call(position_embeddings)
    return _tc_call(position_embeddings, sc_out)


def kernel(inputs, position_embeddings):
    return _pe_full(position_embeddings).reshape(inputs.shape)


# speedup vs baseline: 1.0054x; 1.0054x over previous
import functools

import jax
import jax.numpy as jnp
from jax import lax
from jax.experimental import pallas as pl
from jax.experimental.pallas import tpu as pltpu
from jax.experimental.pallas import tpu_sc as plsc

ALPHA = 0.4
SEQ = 2048
FEAT = 768
BATCH = 4
CHUNKS = 4
NC, NS = 2, 16
NW = NC * NS
ROWS_W = SEQ // NW
HALF = ROWS_W // 2
QTR = 16
NT = FEAT // 16
OUT_ROWS = BATCH * CHUNKS * SEQ


def _build_sc_call():
    mesh = plsc.VectorSubcoreMesh(core_axis_name="c", subcore_axis_name="s")

    @functools.partial(
        pl.kernel,
        mesh=mesh,
        out_type=jax.ShapeDtypeStruct((BATCH * CHUNKS, SEQ, FEAT), jnp.float32),
        scratch_types=[
            pltpu.VMEM((HALF, FEAT), jnp.float32),
            pltpu.VMEM((HALF, FEAT), jnp.float32),
            pltpu.VMEM((4, FEAT), jnp.float32),
            pltpu.VMEM((4, FEAT), jnp.float32),
            pltpu.VMEM((2, QTR, FEAT), jnp.float32),
            pltpu.SemaphoreType.DMA,
            pltpu.SemaphoreType.DMA,
            pltpu.SemaphoreType.DMA,
            pltpu.SemaphoreType.DMA,
        ],
    )
    def pe_kernel(table, out, g0, g1, head, cq, bufs, si, s0, sa, sb):
        w = lax.axis_index("s") * NC + lax.axis_index("c")
        j0 = w * ROWS_W
        cp_head = pltpu.async_copy(table.at[pl.ds(0, 4), :], head, si)
        cp_g0 = pltpu.async_copy(table.at[pl.ds(j0, HALF), :], g0, si)
        cp_g1 = pltpu.async_copy(table.at[pl.ds(j0 + HALF, HALF), :], g1, si)

        a = ALPHA
        s = a / (1.0 - a)
        cp_head.wait()
        cp_g0.wait()
        cp_q0a = pltpu.async_copy(g0, out.at[0, pl.ds(j0, HALF), :], s0)
        for t in range(NT):
            sl = pl.ds(t * 16, 16)
            p0 = head[0, sl]
            pq = head[1, sl]
            cq[1, sl] = s * (pq - a * p0) - a * p0

        gsem = (sa, sb)
        pending = {}
        cp_q0b = None
        for ph in range(2 * HALF // QTR):
            seg = g0 if ph * QTR < HALF else g1
            r0 = ph * QTR % HALF
            gen = ph % 2
            if ph * QTR == HALF:
                cp_g1.wait()
                cp_q0b = pltpu.async_copy(
                    g1, out.at[0, pl.ds(j0 + HALF, HALF), :], s0
                )
            if ph >= 2:
                pending[ph - 2].wait()
            buf = bufs.at[gen]

            @plsc.parallel_loop(0, NT)
            def body(t, _seg=seg, _buf=buf, _r0=r0):
                sl = pl.ds(t * 16, 16)
                c1 = cq[1, sl]
                for r in range(QTR):
                    _buf[r, sl] = _seg[_r0 + r, sl] + c1

            pending[ph] = pltpu.async_copy(
                buf, out.at[1, pl.ds(j0 + ph * QTR, QTR), :], gsem[gen]
            )
        cp_q0a.wait()
        cp_q0b.wait()
        pending[2].wait()
        pending[3].wait()

    return pe_kernel


def _tc_body(table_ref, big_ref, out_ref):
    q = lax.rem(pl.program_id(0) + 2, CHUNKS)
    t = table_ref[...]
    a = ALPHA
    s = a / (1.0 - a)
    p0 = t[0:1, :]
    pq = (
        jnp.where(q == 0, t[0:1, :], 0.0)
        + jnp.where(q == 1, t[1:2, :], 0.0)
        + jnp.where(q == 2, t[2:3, :], 0.0)
        + jnp.where(q == 3, t[3:4, :], 0.0)
    )
    c = s * (pq - a * p0) - a * p0
    out_ref[0] = t + c


def _build_tc_call():
    nblk = BATCH * CHUNKS
    return pl.pallas_call(
        _tc_body,
        grid=(BATCH * CHUNKS - 2,),
        in_specs=[
            pl.BlockSpec((SEQ, FEAT), lambda k: (0, 0)),
            pl.BlockSpec(memory_space=pl.ANY),
        ],
        out_specs=pl.BlockSpec((1, SEQ, FEAT), lambda k: (k + 2, 0, 0)),
        out_shape=jax.ShapeDtypeStruct((nblk, SEQ, FEAT), jnp.float32),
        input_output_aliases={1: 0},
    )


_sc_call = _build_sc_call()
_tc_call = _build_tc_call()


@jax.jit
def _pe_full(position_embeddings):
    sc_out = _sc_call(position_embeddings)
    return _tc_call(position_embeddings, sc_out)


def kernel(inputs, position_embeddings):
    return _pe_full(position_embeddings).reshape(inputs.shape)
